# baseline (speedup 1.0000x reference)
import functools

import jax
import jax.numpy as jnp
from jax import lax
from jax.experimental import pallas as pl
from jax.experimental.pallas import tpu as pltpu
from jax.experimental.pallas import tpu_sc as plsc

L = 16
NC, NS = 2, 16
NW = NC * NS
NSLAB = 25
UNROLL = 25

_GATHER_DNUMS = lax.GatherDimensionNumbers(
    offset_dims=(), collapsed_slice_dims=(0,), start_index_map=(0,))


def _take(x, idx):
    return lax.gather(
        x, idx[:, None], dimension_numbers=_GATHER_DNUMS, slice_sizes=(1,),
        mode=lax.GatherScatterMode.PROMISE_IN_BOUNDS)


def _make_tile_kernel(chunk, slab):
    vps = slab // L
    groups = vps // UNROLL
    mesh = plsc.VectorSubcoreMesh(core_axis_name="c", subcore_axis_name="s")

    @functools.partial(
        pl.kernel,
        out_type=[
            jax.ShapeDtypeStruct((NW * 5 * L,), jnp.float32),
            jax.ShapeDtypeStruct((NW * 3 * L,), jnp.int32),
        ],
        mesh=mesh,
        scratch_types=[
            pltpu.VMEM((slab,), jnp.float32),
            pltpu.VMEM((slab,), jnp.float32),
            pltpu.VMEM((slab + L,), jnp.int32),
            pltpu.VMEM((slab,), jnp.float32),
            pltpu.VMEM((slab,), jnp.float32),
            pltpu.VMEM((slab + L,), jnp.int32),
            pltpu.VMEM((chunk + L,), jnp.float32),
            pltpu.VMEM((chunk + L,), jnp.float32),
            pltpu.VMEM((5 * L,), jnp.float32),
            pltpu.VMEM((3 * L,), jnp.int32),
            pltpu.SemaphoreType.DMA,
            pltpu.SemaphoreType.DMA,
        ],
        compiler_params=pltpu.CompilerParams(
            needs_layout_passes=False, skip_device_barrier=True),
    )
    def tile_kernel(in_hbm, tgt_hbm, ids_hbm, outf_hbm, outi_hbm,
                    bin0, btgt0, bids0, bin1, btgt1, bids1,
                    endpa, endpb, rowf_v, rowi_v, sem0, sem1):
        wid = lax.axis_index("c") * NS + lax.axis_index("s")
        base = wid * chunk

        iota = lax.iota(jnp.int32, L)
        zeros_i = jnp.zeros((L,), jnp.int32)
        ones_i = jnp.full((L,), 1, jnp.int32)
        fifteens = jnp.full((L,), L - 1, jnp.int32)
        f0 = jnp.zeros((L,), jnp.float32)

        bufs = [(bin0, btgt0, bids0), (bin1, btgt1, bids1)]
        sems = [sem0, sem1]

        def issue(s, p):
            off = base + s * slab
            bi, bt, bd = bufs[p]
            return (
                pltpu.async_copy(in_hbm.at[pl.ds(off, slab)], bi, sems[p]),
                pltpu.async_copy(tgt_hbm.at[pl.ds(off, slab)], bt, sems[p]),
                pltpu.async_copy(ids_hbm.at[pl.ds(off, slab)], bd.at[pl.ds(L, slab)],
                                 sems[p]),
            )

        endpa[pl.ds(0, L)] = f0
        endpb[pl.ds(0, L)] = f0

        def make_gbody(bi, bt, bd):
          def gbody(g, c):
            cpa, cpb, rixc, acc = c
            j0 = g * UNROLL
            ns, lcr, lca, lcb, cnt, ta, tb, av, bv = \
                [], [], [], [], [], [], [], [], []
            for k in range(UNROLL):
                idv = bd[pl.ds(L + (j0 + k) * L, L)]
                sid = bd[pl.ds(L - 1 + (j0 + k) * L, L)]
                a = bi[pl.ds((j0 + k) * L, L)]
                b = bt[pl.ds((j0 + k) * L, L)]
                n = jnp.where(idv != sid, 1, 0)
                ns.append(n)
                lcr.append(plsc.cumsum(n))
                lca.append(plsc.cumsum(a))
                lcb.append(plsc.cumsum(b))
                cnt.append(_take(lcr[k], fifteens))
                ta.append(_take(lca[k], fifteens))
                tb.append(_take(lcb[k], fifteens))
                av.append(a)
                bv.append(b)
                d = jnp.abs(a) - jnp.abs(b)
                acc = acc + d * d
            rb, ab, bb = rixc, cpa, cpb
            for k in range(UNROLL):
                rix = lcr[k] + rb
                pa = lca[k] + ab
                pb = lcb[k] + bb
                mask = ns[k] != 0
                plsc.store_scatter(endpa, [rix], pa - av[k], mask=mask)
                plsc.store_scatter(endpb, [rix], pb - bv[k], mask=mask)
                rb = rb + cnt[k]
                ab = ab + ta[k]
                bb = bb + tb[k]
            return ab, bb, rb, acc
          return gbody

        def drain(p):
            bi, bt, bd = bufs[p]
            pltpu.make_async_copy(in_hbm.at[pl.ds(0, slab)], bi, sems[p]).wait()
            pltpu.make_async_copy(tgt_hbm.at[pl.ds(0, slab)], bt, sems[p]).wait()
            pltpu.make_async_copy(ids_hbm.at[pl.ds(0, slab)],
                                  bd.at[pl.ds(L, slab)], sems[p]).wait()

        def issue_dyn(sl, p):
            off = base + sl * slab
            bi, bt, bd = bufs[p]
            pltpu.async_copy(in_hbm.at[pl.ds(off, slab)], bi, sems[p])
            pltpu.async_copy(tgt_hbm.at[pl.ds(off, slab)], bt, sems[p])
            pltpu.async_copy(ids_hbm.at[pl.ds(off, slab)], bd.at[pl.ds(L, slab)],
                             sems[p])

        def slab_step(p, nxt, c):
            cpa, cpb, rixc, acc, pid = c
            bi, bt, bd = bufs[p]
            drain(p)
            bd[pl.ds(0, L)] = pid
            nv, cond = nxt

            @pl.when(cond)
            def _():
                issue_dyn(nv, 1 - p)

            cpa, cpb, rixc, acc = lax.fori_loop(
                0, groups, make_gbody(bi, bt, bd), (cpa, cpb, rixc, acc))
            pid = _take(bd[pl.ds(slab, L)], fifteens)
            return cpa, cpb, rixc, acc, pid

        issue(0, 0)
        bi0, bt0, bd0 = bufs[0]
        drain(0)
        ids0 = bd0[pl.ds(L, L)]
        head_id_v = _take(ids0, zeros_i)
        bd0[pl.ds(0, L)] = head_id_v
        issue(1, 1)
        cpa, cpb, rixc, acc = lax.fori_loop(
            0, groups, make_gbody(bi0, bt0, bd0), (f0, f0, zeros_i, f0))
        pid = _take(bd0[pl.ds(slab, L)], fifteens)
        carry = (cpa, cpb, rixc, acc, pid)

        npairs = (NSLAB - 1) // 2
        true_p = jnp.bool_(True)

        def pair_body(t, c):
            s_odd = 2 * t + 1
            c = slab_step(1, (s_odd + 1, true_p), c)
            c = slab_step(0, (s_odd + 2, s_odd + 2 <= NSLAB - 1), c)
            return c

        carry = lax.fori_loop(0, npairs, pair_body, carry)
        cpa, cpb, rixc, acc, pid = carry
        tid = pid
        k_s = jnp.squeeze(lax.slice(rixc, (0,), (1,)))
        kv = rixc

        def body2(j, acc2):
            ea = endpa[pl.ds(j * L, L)]
            e1a = endpa[pl.ds(j * L + 1, L)]
            eb = endpb[pl.ds(j * L, L)]
            e1b = endpb[pl.ds(j * L + 1, L)]
            qv = iota + j * L
            valid = (qv >= 1) & (qv <= kv - 1)
            da = e1a - ea
            db = e1b - eb
            t = da * da - db * db
            return acc2 + jnp.where(valid, t * t, 0.0)

        acc = lax.fori_loop(0, (k_s + L - 1) // L, body2, acc)

        hbv = kv > 0
        e1a = _take(endpa[pl.ds(0, L)], ones_i)
        e1b = _take(endpb[pl.ds(0, L)], ones_i)
        eka = _take(endpa[pl.ds(k_s, L)], zeros_i)
        ekb = _take(endpb[pl.ds(k_s, L)], zeros_i)
        hin = jnp.where(hbv, e1a, cpa)
        htg = jnp.where(hbv, e1b, cpb)
        tin = jnp.where(hbv, cpa - eka, cpa)
        ttg = jnp.where(hbv, cpb - ekb, cpb)

        acc = _take(plsc.cumsum(acc), fifteens)
        rowf_v[pl.ds(0 * L, L)] = acc
        rowf_v[pl.ds(1 * L, L)] = hin
        rowf_v[pl.ds(2 * L, L)] = htg
        rowf_v[pl.ds(3 * L, L)] = tin
        rowf_v[pl.ds(4 * L, L)] = ttg
        pltpu.sync_copy(rowf_v, outf_hbm.at[pl.ds(wid * 5 * L, 5 * L)])
        rowi_v[pl.ds(0 * L, L)] = head_id_v
        rowi_v[pl.ds(1 * L, L)] = tid
        rowi_v[pl.ds(2 * L, L)] = jnp.where(hbv, 1, 0)
        pltpu.sync_copy(rowi_v, outi_hbm.at[pl.ds(wid * 3 * L, 3 * L)])

    return tile_kernel


def _make_combine_kernel():
    def body(f_ref, i_ref, o_ref):
        def wbody(w, c):
            acc, cid, cin, ctg = c
            fo, io = w * 5 * L, w * 3 * L
            acc = acc + f_ref[fo]
            hin = f_ref[fo + L]
            htg = f_ref[fo + 2 * L]
            tin = f_ref[fo + 3 * L]
            ttg = f_ref[fo + 4 * L]
            hid = i_ref[io]
            tid = i_ref[io + L]
            hbw = i_ref[io + 2 * L] != 0
            merged = hid == cid
            t = cin * cin - ctg * ctg
            acc = acc + jnp.where(merged, 0.0, t * t)
            cin = jnp.where(merged, cin + hin, hin)
            ctg = jnp.where(merged, ctg + htg, htg)
            cid = hid
            t2 = cin * cin - ctg * ctg
            acc = acc + jnp.where(hbw, t2 * t2, 0.0)
            cid = jnp.where(hbw, tid, cid)
            cin = jnp.where(hbw, tin, cin)
            ctg = jnp.where(hbw, ttg, ctg)
            return acc, cid, cin, ctg

        acc, cid, cin, ctg = lax.fori_loop(
            0, NW, wbody,
            (jnp.float32(0.0), jnp.int32(-1), jnp.float32(0.0),
             jnp.float32(0.0)))
        t = cin * cin - ctg * ctg
        o_ref[0] = acc + t * t

    return pl.pallas_call(
        body,
        in_specs=[
            pl.BlockSpec(memory_space=pltpu.SMEM),
            pl.BlockSpec(memory_space=pltpu.SMEM),
        ],
        out_specs=pl.BlockSpec(memory_space=pltpu.SMEM),
        out_shape=jax.ShapeDtypeStruct((1,), jnp.float32),
    )


def kernel(input, target, batch, batch_size):
    n = input.shape[0]
    ids = batch.astype(jnp.int32)
    quantum = NW * NSLAB * UNROLL * L
    n_pad = -n % quantum
    if n_pad:
        input = jnp.concatenate([input, jnp.zeros((n_pad,), input.dtype)])
        target = jnp.concatenate([target, jnp.zeros((n_pad,), target.dtype)])
        ids = jnp.concatenate([ids, jnp.broadcast_to(ids[-1], (n_pad,))])
        n = n + n_pad
    chunk = n // NW
    slab = chunk // NSLAB
    outf, outi = _make_tile_kernel(chunk, slab)(input, target, ids)
    res = _make_combine_kernel()(outf, outi)
    return res[0]


# speedup vs baseline: 1.0009x; 1.0009x over previous
import functools

import jax
import jax.numpy as jnp
from jax import lax
from jax.experimental import pallas as pl
from jax.experimental.pallas import tpu as pltpu
from jax.experimental.pallas import tpu_sc as plsc

L = 16
NC, NS = 2, 16
NW = NC * NS
NSLAB = 25
UNROLL = 25

_GATHER_DNUMS = lax.GatherDimensionNumbers(
    offset_dims=(), collapsed_slice_dims=(0,), start_index_map=(0,))


def _take(x, idx):
    return lax.gather(
        x, idx[:, None], dimension_numbers=_GATHER_DNUMS, slice_sizes=(1,),
        mode=lax.GatherScatterMode.PROMISE_IN_BOUNDS)


def _make_tile_kernel(chunk, slab):
    vps = slab // L
    groups = vps // UNROLL
    mesh = plsc.VectorSubcoreMesh(core_axis_name="c", subcore_axis_name="s")

    @functools.partial(
        pl.kernel,
        out_type=[
            jax.ShapeDtypeStruct((NW * 5 * L,), jnp.float32),
            jax.ShapeDtypeStruct((NW * 3 * L,), jnp.int32),
        ],
        mesh=mesh,
        scratch_types=[
            pltpu.VMEM((slab,), jnp.float32),
            pltpu.VMEM((slab,), jnp.float32),
            pltpu.VMEM((slab + L,), jnp.int32),
            pltpu.VMEM((slab,), jnp.float32),
            pltpu.VMEM((slab,), jnp.float32),
            pltpu.VMEM((slab + L,), jnp.int32),
            pltpu.VMEM((chunk + L,), jnp.float32),
            pltpu.VMEM((chunk + L,), jnp.float32),
            pltpu.VMEM((5 * L,), jnp.float32),
            pltpu.VMEM((3 * L,), jnp.int32),
            pltpu.SemaphoreType.DMA,
            pltpu.SemaphoreType.DMA,
        ],
        compiler_params=pltpu.CompilerParams(needs_layout_passes=False),
    )
    def tile_kernel(in_hbm, tgt_hbm, ids_hbm, outf_hbm, outi_hbm,
                    bin0, btgt0, bids0, bin1, btgt1, bids1,
                    endpa, endpb, rowf_v, rowi_v, sem0, sem1):
        wid = lax.axis_index("c") * NS + lax.axis_index("s")
        base = wid * chunk

        iota = lax.iota(jnp.int32, L)
        zeros_i = jnp.zeros((L,), jnp.int32)
        ones_i = jnp.full((L,), 1, jnp.int32)
        fifteens = jnp.full((L,), L - 1, jnp.int32)
        f0 = jnp.zeros((L,), jnp.float32)

        bufs = [(bin0, btgt0, bids0), (bin1, btgt1, bids1)]
        sems = [sem0, sem1]

        def issue(s, p):
            off = base + s * slab
            bi, bt, bd = bufs[p]
            return (
                pltpu.async_copy(in_hbm.at[pl.ds(off, slab)], bi, sems[p]),
                pltpu.async_copy(tgt_hbm.at[pl.ds(off, slab)], bt, sems[p]),
                pltpu.async_copy(ids_hbm.at[pl.ds(off, slab)], bd.at[pl.ds(L, slab)],
                                 sems[p]),
            )

        endpa[pl.ds(0, L)] = f0
        endpb[pl.ds(0, L)] = f0

        def make_gbody(bi, bt, bd):
          def gbody(g, c):
            cpa, cpb, rixc, acc = c
            j0 = g * UNROLL
            ns, lcr, lca, lcb, cnt, ta, tb, av, bv = \
                [], [], [], [], [], [], [], [], []
            for k in range(UNROLL):
                idv = bd[pl.ds(L + (j0 + k) * L, L)]
                sid = bd[pl.ds(L - 1 + (j0 + k) * L, L)]
                a = bi[pl.ds((j0 + k) * L, L)]
                b = bt[pl.ds((j0 + k) * L, L)]
                n = jnp.where(idv != sid, 1, 0)
                ns.append(n)
                lcr.append(plsc.cumsum(n))
                lca.append(plsc.cumsum(a))
                lcb.append(plsc.cumsum(b))
                cnt.append(_take(lcr[k], fifteens))
                ta.append(_take(lca[k], fifteens))
                tb.append(_take(lcb[k], fifteens))
                av.append(a)
                bv.append(b)
                d = jnp.abs(a) - jnp.abs(b)
                acc = acc + d * d
            rb, ab, bb = rixc, cpa, cpb
            for k in range(UNROLL):
                rix = lcr[k] + rb
                pa = lca[k] + ab
                pb = lcb[k] + bb
                mask = ns[k] != 0
                plsc.store_scatter(endpa, [rix], pa - av[k], mask=mask)
                plsc.store_scatter(endpb, [rix], pb - bv[k], mask=mask)
                rb = rb + cnt[k]
                ab = ab + ta[k]
                bb = bb + tb[k]
            return ab, bb, rb, acc
          return gbody

        def drain(p):
            bi, bt, bd = bufs[p]
            pltpu.make_async_copy(in_hbm.at[pl.ds(0, slab)], bi, sems[p]).wait()
            pltpu.make_async_copy(tgt_hbm.at[pl.ds(0, slab)], bt, sems[p]).wait()
            pltpu.make_async_copy(ids_hbm.at[pl.ds(0, slab)],
                                  bd.at[pl.ds(L, slab)], sems[p]).wait()

        def issue_dyn(sl, p):
            off = base + sl * slab
            bi, bt, bd = bufs[p]
            pltpu.async_copy(in_hbm.at[pl.ds(off, slab)], bi, sems[p])
            pltpu.async_copy(tgt_hbm.at[pl.ds(off, slab)], bt, sems[p])
            pltpu.async_copy(ids_hbm.at[pl.ds(off, slab)], bd.at[pl.ds(L, slab)],
                             sems[p])

        def slab_step(p, nxt, c):
            cpa, cpb, rixc, acc, pid = c
            bi, bt, bd = bufs[p]
            drain(p)
            bd[pl.ds(0, L)] = pid
            nv, cond = nxt

            @pl.when(cond)
            def _():
                issue_dyn(nv, 1 - p)

            cpa, cpb, rixc, acc = lax.fori_loop(
                0, groups, make_gbody(bi, bt, bd), (cpa, cpb, rixc, acc))
            pid = _take(bd[pl.ds(slab, L)], fifteens)
            return cpa, cpb, rixc, acc, pid

        issue(0, 0)
        bi0, bt0, bd0 = bufs[0]
        drain(0)
        ids0 = bd0[pl.ds(L, L)]
        head_id_v = _take(ids0, zeros_i)
        bd0[pl.ds(0, L)] = head_id_v
        issue(1, 1)
        cpa, cpb, rixc, acc = lax.fori_loop(
            0, groups, make_gbody(bi0, bt0, bd0), (f0, f0, zeros_i, f0))
        pid = _take(bd0[pl.ds(slab, L)], fifteens)
        carry = (cpa, cpb, rixc, acc, pid)

        npairs = (NSLAB - 1) // 2
        true_p = jnp.bool_(True)

        def pair_body(t, c):
            s_odd = 2 * t + 1
            c = slab_step(1, (s_odd + 1, true_p), c)
            c = slab_step(0, (s_odd + 2, s_odd + 2 <= NSLAB - 1), c)
            return c

        carry = lax.fori_loop(0, npairs, pair_body, carry)
        cpa, cpb, rixc, acc, pid = carry
        tid = pid
        k_s = jnp.squeeze(lax.slice(rixc, (0,), (1,)))
        kv = rixc

        def body2(j, acc2):
            ea = endpa[pl.ds(j * L, L)]
            e1a = endpa[pl.ds(j * L + 1, L)]
            eb = endpb[pl.ds(j * L, L)]
            e1b = endpb[pl.ds(j * L + 1, L)]
            qv = iota + j * L
            valid = (qv >= 1) & (qv <= kv - 1)
            da = e1a - ea
            db = e1b - eb
            t = da * da - db * db
            return acc2 + jnp.where(valid, t * t, 0.0)

        acc = lax.fori_loop(0, (k_s + L - 1) // L, body2, acc)

        hbv = kv > 0
        e1a = _take(endpa[pl.ds(0, L)], ones_i)
        e1b = _take(endpb[pl.ds(0, L)], ones_i)
        eka = _take(endpa[pl.ds(k_s, L)], zeros_i)
        ekb = _take(endpb[pl.ds(k_s, L)], zeros_i)
        hin = jnp.where(hbv, e1a, cpa)
        htg = jnp.where(hbv, e1b, cpb)
        tin = jnp.where(hbv, cpa - eka, cpa)
        ttg = jnp.where(hbv, cpb - ekb, cpb)

        acc = _take(plsc.cumsum(acc), fifteens)
        rowf_v[pl.ds(0 * L, L)] = acc
        rowf_v[pl.ds(1 * L, L)] = hin
        rowf_v[pl.ds(2 * L, L)] = htg
        rowf_v[pl.ds(3 * L, L)] = tin
        rowf_v[pl.ds(4 * L, L)] = ttg
        pltpu.sync_copy(rowf_v, outf_hbm.at[pl.ds(wid * 5 * L, 5 * L)])
        rowi_v[pl.ds(0 * L, L)] = head_id_v
        rowi_v[pl.ds(1 * L, L)] = tid
        rowi_v[pl.ds(2 * L, L)] = jnp.where(hbv, 1, 0)
        pltpu.sync_copy(rowi_v, outi_hbm.at[pl.ds(wid * 3 * L, 3 * L)])

    return tile_kernel


def _make_combine_kernel():
    def body(f_ref, i_ref, o_ref):
        def wbody(w, c):
            acc, cid, cin, ctg = c
            fo, io = w * 5 * L, w * 3 * L
            acc = acc + f_ref[fo]
            hin = f_ref[fo + L]
            htg = f_ref[fo + 2 * L]
            tin = f_ref[fo + 3 * L]
            ttg = f_ref[fo + 4 * L]
            hid = i_ref[io]
            tid = i_ref[io + L]
            hbw = i_ref[io + 2 * L] != 0
            merged = hid == cid
            t = cin * cin - ctg * ctg
            acc = acc + jnp.where(merged, 0.0, t * t)
            cin = jnp.where(merged, cin + hin, hin)
            ctg = jnp.where(merged, ctg + htg, htg)
            cid = hid
            t2 = cin * cin - ctg * ctg
            acc = acc + jnp.where(hbw, t2 * t2, 0.0)
            cid = jnp.where(hbw, tid, cid)
            cin = jnp.where(hbw, tin, cin)
            ctg = jnp.where(hbw, ttg, ctg)
            return acc, cid, cin, ctg

        acc, cid, cin, ctg = lax.fori_loop(
            0, NW, wbody,
            (jnp.float32(0.0), jnp.int32(-1), jnp.float32(0.0),
             jnp.float32(0.0)))
        t = cin * cin - ctg * ctg
        o_ref[0] = acc + t * t

    return pl.pallas_call(
        body,
        in_specs=[
            pl.BlockSpec(memory_space=pltpu.SMEM),
            pl.BlockSpec(memory_space=pltpu.SMEM),
        ],
        out_specs=pl.BlockSpec(memory_space=pltpu.SMEM),
        out_shape=jax.ShapeDtypeStruct((1,), jnp.float32),
    )


def kernel(input, target, batch, batch_size):
    n = input.shape[0]
    ids = batch.astype(jnp.int32)
    quantum = NW * NSLAB * UNROLL * L
    n_pad = -n % quantum
    if n_pad:
        input = jnp.concatenate([input, jnp.zeros((n_pad,), input.dtype)])
        target = jnp.concatenate([target, jnp.zeros((n_pad,), target.dtype)])
        ids = jnp.concatenate([ids, jnp.broadcast_to(ids[-1], (n_pad,))])
        n = n + n_pad
    chunk = n // NW
    slab = chunk // NSLAB
    outf, outi = _make_tile_kernel(chunk, slab)(input, target, ids)
    res = _make_combine_kernel()(outf, outi)
    return res[0]


# speedup vs baseline: 1.0089x; 1.0080x over previous
import functools

import jax
import jax.numpy as jnp
from jax import lax
from jax.experimental import pallas as pl
from jax.experimental.pallas import tpu as pltpu
from jax.experimental.pallas import tpu_sc as plsc

L = 16
NC, NS = 2, 16
NW = NC * NS
NSLAB = 25
UNROLL = 25

_GATHER_DNUMS = lax.GatherDimensionNumbers(
    offset_dims=(), collapsed_slice_dims=(0,), start_index_map=(0,))


def _take(x, idx):
    return lax.gather(
        x, idx[:, None], dimension_numbers=_GATHER_DNUMS, slice_sizes=(1,),
        mode=lax.GatherScatterMode.PROMISE_IN_BOUNDS)


def _make_tile_kernel(chunk, slab):
    vps = slab // L
    groups = vps // UNROLL
    mesh = plsc.VectorSubcoreMesh(core_axis_name="c", subcore_axis_name="s")

    @functools.partial(
        pl.kernel,
        out_type=[
            jax.ShapeDtypeStruct((NW * 5 * L,), jnp.float32),
            jax.ShapeDtypeStruct((NW * 3 * L,), jnp.int32),
        ],
        mesh=mesh,
        scratch_types=[
            pltpu.VMEM((slab,), jnp.float32),
            pltpu.VMEM((slab,), jnp.float32),
            pltpu.VMEM((slab + L,), jnp.int32),
            pltpu.VMEM((slab,), jnp.float32),
            pltpu.VMEM((slab,), jnp.float32),
            pltpu.VMEM((slab + L,), jnp.int32),
            pltpu.VMEM((chunk + L,), jnp.float32),
            pltpu.VMEM((chunk + L,), jnp.float32),
            pltpu.VMEM((5 * L,), jnp.float32),
            pltpu.VMEM((3 * L,), jnp.int32),
            pltpu.SemaphoreType.DMA,
            pltpu.SemaphoreType.DMA,
        ],
        compiler_params=pltpu.CompilerParams(needs_layout_passes=False),
    )
    def tile_kernel(in_hbm, tgt_hbm, ids_hbm, outf_hbm, outi_hbm,
                    bin0, btgt0, bids0, bin1, btgt1, bids1,
                    endpa, endpb, rowf_v, rowi_v, sem0, sem1):
        wid = lax.axis_index("c") * NS + lax.axis_index("s")
        base = wid * chunk

        iota = lax.iota(jnp.int32, L)
        zeros_i = jnp.zeros((L,), jnp.int32)
        ones_i = jnp.full((L,), 1, jnp.int32)
        fifteens = jnp.full((L,), L - 1, jnp.int32)
        f0 = jnp.zeros((L,), jnp.float32)

        bufs = [(bin0, btgt0, bids0), (bin1, btgt1, bids1)]
        sems = [sem0, sem1]

        def issue(s, p):
            off = base + s * slab
            bi, bt, bd = bufs[p]
            return (
                pltpu.async_copy(in_hbm.at[pl.ds(off, slab)], bi, sems[p]),
                pltpu.async_copy(tgt_hbm.at[pl.ds(off, slab)], bt, sems[p]),
                pltpu.async_copy(ids_hbm.at[pl.ds(off, slab)], bd.at[pl.ds(L, slab)],
                                 sems[p]),
            )

        endpa[pl.ds(0, L)] = f0
        endpb[pl.ds(0, L)] = f0

        def make_gbody(bi, bt, bd):
          def gbody(g, c):
            cpa, cpb, off, acc = c
            j0 = g * UNROLL
            ms, lca, lcb, cnt, ta, tb, av, bv = [], [], [], [], [], [], [], []
            for k in range(UNROLL):
                idv = bd[pl.ds(L + (j0 + k) * L, L)]
                sid = bd[pl.ds(L - 1 + (j0 + k) * L, L)]
                a = bi[pl.ds((j0 + k) * L, L)]
                b = bt[pl.ds((j0 + k) * L, L)]
                mask = idv != sid
                ms.append(mask)
                lca.append(plsc.cumsum(a))
                lcb.append(plsc.cumsum(b))
                cntv = plsc.all_reduce_population_count(mask)
                cnt.append(jnp.squeeze(lax.slice(cntv, (0,), (1,))))
                ta.append(_take(lca[k], fifteens))
                tb.append(_take(lcb[k], fifteens))
                av.append(a)
                bv.append(b)
                d = jnp.abs(a) - jnp.abs(b)
                acc = acc + d * d
            ab, bb = cpa, cpb
            for k in range(UNROLL):
                pa = lca[k] + ab
                pb = lcb[k] + bb
                plsc.store_compressed(endpa.at[pl.ds(off + 1, L)],
                                      pa - av[k], mask=ms[k])
                plsc.store_compressed(endpb.at[pl.ds(off + 1, L)],
                                      pb - bv[k], mask=ms[k])
                off = off + cnt[k]
                ab = ab + ta[k]
                bb = bb + tb[k]
            return ab, bb, off, acc
          return gbody

        def drain(p):
            bi, bt, bd = bufs[p]
            pltpu.make_async_copy(in_hbm.at[pl.ds(0, slab)], bi, sems[p]).wait()
            pltpu.make_async_copy(tgt_hbm.at[pl.ds(0, slab)], bt, sems[p]).wait()
            pltpu.make_async_copy(ids_hbm.at[pl.ds(0, slab)],
                                  bd.at[pl.ds(L, slab)], sems[p]).wait()

        def issue_dyn(sl, p):
            off = base + sl * slab
            bi, bt, bd = bufs[p]
            pltpu.async_copy(in_hbm.at[pl.ds(off, slab)], bi, sems[p])
            pltpu.async_copy(tgt_hbm.at[pl.ds(off, slab)], bt, sems[p])
            pltpu.async_copy(ids_hbm.at[pl.ds(off, slab)], bd.at[pl.ds(L, slab)],
                             sems[p])

        def slab_step(p, nxt, c):
            cpa, cpb, rixc, acc, pid = c
            bi, bt, bd = bufs[p]
            drain(p)
            bd[pl.ds(0, L)] = pid
            nv, cond = nxt

            @pl.when(cond)
            def _():
                issue_dyn(nv, 1 - p)

            cpa, cpb, rixc, acc = lax.fori_loop(
                0, groups, make_gbody(bi, bt, bd), (cpa, cpb, rixc, acc))
            pid = _take(bd[pl.ds(slab, L)], fifteens)
            return cpa, cpb, rixc, acc, pid

        issue(0, 0)
        bi0, bt0, bd0 = bufs[0]
        drain(0)
        ids0 = bd0[pl.ds(L, L)]
        head_id_v = _take(ids0, zeros_i)
        bd0[pl.ds(0, L)] = head_id_v
        issue(1, 1)
        cpa, cpb, rixc, acc = lax.fori_loop(
            0, groups, make_gbody(bi0, bt0, bd0), (f0, f0, jnp.int32(0), f0))
        pid = _take(bd0[pl.ds(slab, L)], fifteens)
        carry = (cpa, cpb, rixc, acc, pid)

        npairs = (NSLAB - 1) // 2
        true_p = jnp.bool_(True)

        def pair_body(t, c):
            s_odd = 2 * t + 1
            c = slab_step(1, (s_odd + 1, true_p), c)
            c = slab_step(0, (s_odd + 2, s_odd + 2 <= NSLAB - 1), c)
            return c

        carry = lax.fori_loop(0, npairs, pair_body, carry)
        cpa, cpb, rixc, acc, pid = carry
        tid = pid
        k_s = rixc
        kv = jnp.broadcast_to(k_s, (L,))

        def body2(j, acc2):
            ea = endpa[pl.ds(j * L, L)]
            e1a = endpa[pl.ds(j * L + 1, L)]
            eb = endpb[pl.ds(j * L, L)]
            e1b = endpb[pl.ds(j * L + 1, L)]
            qv = iota + j * L
            valid = (qv >= 1) & (qv <= kv - 1)
            da = e1a - ea
            db = e1b - eb
            t = da * da - db * db
            return acc2 + jnp.where(valid, t * t, 0.0)

        acc = lax.fori_loop(0, (k_s + L - 1) // L, body2, acc)

        hbv = kv > 0
        e1a = _take(endpa[pl.ds(0, L)], ones_i)
        e1b = _take(endpb[pl.ds(0, L)], ones_i)
        eka = _take(endpa[pl.ds(k_s, L)], zeros_i)
        ekb = _take(endpb[pl.ds(k_s, L)], zeros_i)
        hin = jnp.where(hbv, e1a, cpa)
        htg = jnp.where(hbv, e1b, cpb)
        tin = jnp.where(hbv, cpa - eka, cpa)
        ttg = jnp.where(hbv, cpb - ekb, cpb)

        acc = _take(plsc.cumsum(acc), fifteens)
        rowf_v[pl.ds(0 * L, L)] = acc
        rowf_v[pl.ds(1 * L, L)] = hin
        rowf_v[pl.ds(2 * L, L)] = htg
        rowf_v[pl.ds(3 * L, L)] = tin
        rowf_v[pl.ds(4 * L, L)] = ttg
        pltpu.sync_copy(rowf_v, outf_hbm.at[pl.ds(wid * 5 * L, 5 * L)])
        rowi_v[pl.ds(0 * L, L)] = head_id_v
        rowi_v[pl.ds(1 * L, L)] = tid
        rowi_v[pl.ds(2 * L, L)] = jnp.where(hbv, 1, 0)
        pltpu.sync_copy(rowi_v, outi_hbm.at[pl.ds(wid * 3 * L, 3 * L)])

    return tile_kernel


def _make_combine_kernel():
    def body(f_ref, i_ref, o_ref):
        def wbody(w, c):
            acc, cid, cin, ctg = c
            fo, io = w * 5 * L, w * 3 * L
            acc = acc + f_ref[fo]
            hin = f_ref[fo + L]
            htg = f_ref[fo + 2 * L]
            tin = f_ref[fo + 3 * L]
            ttg = f_ref[fo + 4 * L]
            hid = i_ref[io]
            tid = i_ref[io + L]
            hbw = i_ref[io + 2 * L] != 0
            merged = hid == cid
            t = cin * cin - ctg * ctg
            acc = acc + jnp.where(merged, 0.0, t * t)
            cin = jnp.where(merged, cin + hin, hin)
            ctg = jnp.where(merged, ctg + htg, htg)
            cid = hid
            t2 = cin * cin - ctg * ctg
            acc = acc + jnp.where(hbw, t2 * t2, 0.0)
            cid = jnp.where(hbw, tid, cid)
            cin = jnp.where(hbw, tin, cin)
            ctg = jnp.where(hbw, ttg, ctg)
            return acc, cid, cin, ctg

        acc, cid, cin, ctg = lax.fori_loop(
            0, NW, wbody,
            (jnp.float32(0.0), jnp.int32(-1), jnp.float32(0.0),
             jnp.float32(0.0)))
        t = cin * cin - ctg * ctg
        o_ref[0] = acc + t * t

    return pl.pallas_call(
        body,
        in_specs=[
            pl.BlockSpec(memory_space=pltpu.SMEM),
            pl.BlockSpec(memory_space=pltpu.SMEM),
        ],
        out_specs=pl.BlockSpec(memory_space=pltpu.SMEM),
        out_shape=jax.ShapeDtypeStruct((1,), jnp.float32),
    )


def kernel(input, target, batch, batch_size):
    n = input.shape[0]
    ids = batch.astype(jnp.int32)
    quantum = NW * NSLAB * UNROLL * L
    n_pad = -n % quantum
    if n_pad:
        input = jnp.concatenate([input, jnp.zeros((n_pad,), input.dtype)])
        target = jnp.concatenate([target, jnp.zeros((n_pad,), target.dtype)])
        ids = jnp.concatenate([ids, jnp.broadcast_to(ids[-1], (n_pad,))])
        n = n + n_pad
    chunk = n // NW
    slab = chunk // NSLAB
    outf, outi = _make_tile_kernel(chunk, slab)(input, target, ids)
    res = _make_combine_kernel()(outf, outi)
    return res[0]
